# Initial kernel scaffold; baseline (speedup 1.0000x reference)
import jax
import jax.numpy as jnp
from jax.experimental import pallas as pl


def kernel(x, edge_index, raw_affine, W0, a_src0, a_dst0, W1, a_src1, a_dst1, W2, a_src2, a_dst2, W3, a_src3, a_dst3, src_W, src_b, dst_W, dst_b, p1, c1_W, c1_b, p2, c2_W, c2_b):
    raise NotImplementedError("write your pallas kernel here")


# speedup vs baseline: 11.5806x; 11.5806x over previous
import functools

import jax
import jax.numpy as jnp
from jax import lax
from jax.experimental import pallas as pl
from jax.experimental.pallas import tpu as pltpu, tpu_sc as plsc

NC, NS = 2, 16
NW = NC * NS
C = 128
_EPS = 1e-9


def _mesh():
    return plsc.VectorSubcoreMesh(core_axis_name="c", subcore_axis_name="s")


def _tc_stage(mode, din, dout, parts, s2d, W, A2, np_):
    bm = 1024
    grid = (np_ // bm,)

    def body(p_ref, s_ref, W_ref, A2_ref, hw_ref, sv_ref):
        if mode == "first":
            h0 = p_ref[...]
            hw = jnp.dot(h0, W_ref[...], preferred_element_type=jnp.float32)
        elif mode == "fsplit":
            s = s_ref[...] + _EPS
            h0 = p_ref[0] / s
            h1 = p_ref[1] / s
            h0 = jnp.where(h0 > 0, h0, jnp.exp(h0) - 1.0)
            h1 = jnp.where(h1 > 0, h1, jnp.exp(h1) - 1.0)
            hw = jnp.dot(h0, W_ref[0:128, :], preferred_element_type=jnp.float32)
            hw += jnp.dot(h1, W_ref[128:256, :], preferred_element_type=jnp.float32)
        else:
            s = s_ref[...] + _EPS
            h0 = (p_ref[0] + p_ref[1]) / s
            h0 = jnp.where(h0 > 0, h0, jnp.exp(h0) - 1.0)
            hw = jnp.dot(h0, W_ref[...], preferred_element_type=jnp.float32)
        sv_ref[...] = jnp.dot(hw, A2_ref[...], preferred_element_type=jnp.float32)
        if dout == 256:
            hw_ref[0] = hw[:, 0:128]
            hw_ref[1] = hw[:, 128:256]
        else:
            hw_ref[...] = hw

    if mode == "first":
        p_spec = pl.BlockSpec((bm, 128), lambda m: (m, 0))
    else:
        p_spec = pl.BlockSpec((2, bm, 128), lambda m: (0, m, 0))
    if dout == 256:
        hw_shape = jax.ShapeDtypeStruct((2, np_, 128), jnp.float32)
        hw_spec = pl.BlockSpec((2, bm, 128), lambda m: (0, m, 0))
    else:
        hw_shape = jax.ShapeDtypeStruct((np_, 128), jnp.float32)
        hw_spec = pl.BlockSpec((bm, 128), lambda m: (m, 0))

    return pl.pallas_call(
        body,
        grid=grid,
        in_specs=[
            p_spec,
            pl.BlockSpec((bm, 1), lambda m: (m, 0)),
            pl.BlockSpec((din, dout), lambda m: (0, 0)),
            pl.BlockSpec((dout, 2), lambda m: (0, 0)),
        ],
        out_specs=[hw_spec, pl.BlockSpec((bm, 2), lambda m: (m, 0))],
        out_shape=[hw_shape, jax.ShapeDtypeStruct((np_, 2), jnp.float32)],
    )(parts, s2d, W, A2)


def _tc_sfdf(acc, s2d, srcW, srcb, dstW, dstb, np_):
    bm = 1024
    grid = (np_ // bm,)

    def body(p_ref, s_ref, sw_ref, sb_ref, dw_ref, db_ref, sf_ref, df_ref):
        s = s_ref[...] + _EPS
        h = (p_ref[0] + p_ref[1]) / s
        h = jnp.where(h > 0, h, jnp.exp(h) - 1.0)
        sf_ref[...] = jnp.dot(h, sw_ref[...], preferred_element_type=jnp.float32) + sb_ref[...]
        df_ref[...] = jnp.dot(h, dw_ref[...], preferred_element_type=jnp.float32) + db_ref[...]

    io = pl.BlockSpec((bm, 128), lambda m: (m, 0))
    wspec = pl.BlockSpec((128, 128), lambda m: (0, 0))
    bspec = pl.BlockSpec((1, 128), lambda m: (0, 0))
    return pl.pallas_call(
        body,
        grid=grid,
        in_specs=[pl.BlockSpec((2, bm, 128), lambda m: (0, m, 0)),
                  pl.BlockSpec((bm, 1), lambda m: (m, 0)), wspec, bspec, wspec, bspec],
        out_specs=[io, io],
        out_shape=[jax.ShapeDtypeStruct((np_, 128), jnp.float32)] * 2,
    )(acc, s2d, srcW, srcb, dstW, dstb)


def _tc_mlp(z, c1W, c1b, p2, c2W, c2b, ra, e_):
    bm = 640
    grid = (e_ // bm,)

    def body(z_ref, w1_ref, b1_ref, p2_ref, w2_ref, b2_ref, ra_ref, pc_ref, msg_ref):
        u = jnp.dot(z_ref[...], w1_ref[...], preferred_element_type=jnp.float32) + b1_ref[...]
        u = jnp.where(u >= 0, u, p2_ref[...] * u)
        pc = jnp.dot(u, w2_ref[...], preferred_element_type=jnp.float32) + b2_ref[...]
        pc_ref[...] = pc
        p0 = pc[:, 0:1]
        p1c = pc[:, 1:2]
        m = jnp.maximum(p0, p1c)
        e0 = jnp.exp(p0 - m)
        e1 = jnp.exp(p1c - m)
        msg_ref[...] = ra_ref[...] * (e1 - e0) / (e0 + e1)

    return pl.pallas_call(
        body,
        grid=grid,
        in_specs=[
            pl.BlockSpec((bm, 128), lambda m: (m, 0)),
            pl.BlockSpec((128, 128), lambda m: (0, 0)),
            pl.BlockSpec((1, 128), lambda m: (0, 0)),
            pl.BlockSpec((1, 128), lambda m: (0, 0)),
            pl.BlockSpec((128, 2), lambda m: (0, 0)),
            pl.BlockSpec((1, 2), lambda m: (0, 0)),
            pl.BlockSpec((bm, 1), lambda m: (m, 0)),
        ],
        out_specs=[pl.BlockSpec((bm, 2), lambda m: (m, 0)),
                   pl.BlockSpec((bm, 1), lambda m: (m, 0))],
        out_shape=[jax.ShapeDtypeStruct((e_, 2), jnp.float32),
                   jax.ShapeDtypeStruct((e_, 1), jnp.float32)],
    )(z, c1W, c1b, p2, c2W, c2b, ra)


def _tc_den(denp, degp, np_):
    def body(den_ref, deg_ref, o_ref):
        d = den_ref[0:1, :] + den_ref[1:2, :]
        g = deg_ref[0:1, :] + deg_ref[1:2, :]
        o_ref[...] = d / jnp.maximum(g, 1.0)

    return pl.pallas_call(
        body,
        out_shape=jax.ShapeDtypeStruct((1, np_), jnp.float32),
    )(denp, degp)


def _sc_w(ssrc, sdst, src2d, dst2d, np_, nch):
    rpt = np_ // NS
    nbase, rem = nch // NW, nch % NW

    @functools.partial(
        pl.kernel,
        mesh=_mesh(),
        out_type=(
            jax.ShapeDtypeStruct((nch, C), jnp.float32),
            jax.ShapeDtypeStruct((NC, np_), jnp.float32),
        ),
        scratch_types=[
            pltpu.VMEM((C,), jnp.int32),
            pltpu.VMEM((C,), jnp.int32),
            pltpu.VMEM((C,), jnp.float32),
            pltpu.VMEM((C,), jnp.float32),
            pltpu.VMEM((C,), jnp.float32),
            pltpu.MemorySpace.VMEM_SHARED((np_,), jnp.float32),
            pltpu.SemaphoreType.DMA,
            pltpu.SemaphoreType.DMA,
        ],
    )
    def k(ssrc_h, sdst_h, src_h, dst_h, w_h, spart_h, srcc, dstc, vs, vd, wbuf, s_sh, sem, sem2):
        cid = lax.axis_index("c")
        sid = lax.axis_index("s")
        wid = sid * NC + cid
        z16 = jnp.zeros((16,), jnp.float32)
        for j in range(C // 16):
            wbuf[pl.ds(j * 16, 16)] = z16
        for r in range(rpt // C):
            pltpu.sync_copy(wbuf, s_sh.at[pl.ds(sid * rpt + r * C, C)])
        plsc.subcore_barrier()

        nch_w = nbase + jnp.where(wid < rem, 1, 0)

        @pl.loop(0, nch_w)
        def _chunk(i):
            chunk = wid + i * NW
            pltpu.sync_copy(src_h.at[chunk], srcc)
            pltpu.sync_copy(dst_h.at[chunk], dstc)
            cp1 = pltpu.async_copy(ssrc_h.at[srcc], vs, sem)
            cp2 = pltpu.async_copy(sdst_h.at[dstc], vd, sem2)
            cp1.wait()
            cp2.wait()
            for j in range(C // 16):
                e = vs[pl.ds(j * 16, 16)] + vd[pl.ds(j * 16, 16)]
                e = jnp.where(e >= 0.0, e, 0.2 * e)
                wbuf[pl.ds(j * 16, 16)] = jnp.exp(e)
            pltpu.sync_copy(wbuf, w_h.at[chunk])
            pltpu.sync_copy(wbuf, s_sh.at[dstc], add=True)

        plsc.subcore_barrier()
        pltpu.sync_copy(
            s_sh.at[pl.ds(sid * rpt, rpt)],
            spart_h.at[cid, pl.ds(sid * rpt, rpt)],
        )

    return k(ssrc, sdst, src2d, dst2d)


def _sc_agg(table, w2d, src2d, dst2d, fsplit, np_, nch):
    rpt = np_ // NS

    @functools.partial(
        pl.kernel,
        mesh=_mesh(),
        out_type=jax.ShapeDtypeStruct((NC, np_, C), jnp.float32),
        scratch_types=[
            pltpu.VMEM((C,), jnp.int32),
            pltpu.VMEM((C,), jnp.int32),
            pltpu.VMEM((C,), jnp.int32),
            pltpu.VMEM((C,), jnp.float32),
            pltpu.VMEM((C, C), jnp.float32),
            pltpu.MemorySpace.VMEM_SHARED((np_, C), jnp.float32),
            pltpu.SemaphoreType.DMA,
        ],
    )
    def k(tbl_h, w_h, src_h, dst_h, acc_h, srcc, srcc2, dstc, wbuf, rows, acc_sh, sem):
        cid = lax.axis_index("c")
        sid = lax.axis_index("s")
        z16 = jnp.zeros((16,), jnp.float32)

        @pl.loop(0, C)
        def _zr(r):
            for j in range(C // 16):
                rows[r, pl.ds(j * 16, 16)] = z16

        for b in range(rpt // C):
            pltpu.sync_copy(rows, acc_sh.at[pl.ds(sid * rpt + b * C, C)])
        plsc.subcore_barrier()

        if fsplit:
            nbase, rem = nch // NS, nch % NS
            nch_w = nbase + jnp.where(sid < rem, 1, 0)
            first = sid
        else:
            half = nch // NC
            nbase, rem = half // NS, half % NS
            nch_w = nbase + jnp.where(sid < rem, 1, 0)
            first = cid * half + sid

        @pl.loop(0, nch_w)
        def _chunk(i):
            chunk = first + i * NS
            pltpu.sync_copy(src_h.at[chunk], srcc)
            pltpu.sync_copy(dst_h.at[chunk], dstc)
            pltpu.sync_copy(w_h.at[chunk], wbuf)
            if fsplit:
                off = (cid * np_).astype(jnp.int32)
                for j in range(C // 16):
                    srcc2[pl.ds(j * 16, 16)] = srcc[pl.ds(j * 16, 16)] + off
                pltpu.async_copy(tbl_h.at[srcc2], rows, sem).wait()
            else:
                pltpu.async_copy(tbl_h.at[srcc], rows, sem).wait()

            @pl.loop(0, C // 16)
            def _scale(g):
                w16 = wbuf[pl.ds(g * 16, 16)]
                for l in range(16):
                    r = g * 16 + l
                    wr = w16[l]
                    for j in range(C // 16):
                        rows[r, pl.ds(j * 16, 16)] = rows[r, pl.ds(j * 16, 16)] * wr

            pltpu.sync_copy(rows, acc_sh.at[dstc], add=True)

        plsc.subcore_barrier()
        pltpu.sync_copy(
            acc_sh.at[pl.ds(sid * rpt, rpt)],
            acc_h.at[cid, pl.ds(sid * rpt, rpt)],
        )

    return k(table, w2d, src2d, dst2d)


def _sc_z(sf_t, df_t, src2d, dst2d, p1, e_, nch):
    nbase, rem = nch // NW, nch % NW

    @functools.partial(
        pl.kernel,
        mesh=_mesh(),
        out_type=jax.ShapeDtypeStruct((e_, C), jnp.float32),
        scratch_types=[
            pltpu.VMEM((C,), jnp.int32),
            pltpu.VMEM((C,), jnp.int32),
            pltpu.VMEM((C,), jnp.float32),
            pltpu.VMEM((C, C), jnp.float32),
            pltpu.VMEM((C, C), jnp.float32),
            pltpu.SemaphoreType.DMA,
            pltpu.SemaphoreType.DMA,
        ],
    )
    def k(sf_h, df_h, src_h, dst_h, p1_h, z_h, srcc, dstc, p1buf, sfr, dfr, sem, sem2):
        cid = lax.axis_index("c")
        sid = lax.axis_index("s")
        wid = sid * NC + cid
        pltpu.sync_copy(p1_h, p1buf)
        nch_w = nbase + jnp.where(wid < rem, 1, 0)

        @pl.loop(0, nch_w)
        def _chunk(i):
            chunk = wid + i * NW
            pltpu.sync_copy(src_h.at[chunk], srcc)
            pltpu.sync_copy(dst_h.at[chunk], dstc)
            cp1 = pltpu.async_copy(sf_h.at[srcc], sfr, sem)
            cp2 = pltpu.async_copy(df_h.at[dstc], dfr, sem2)
            cp1.wait()
            cp2.wait()

            @pl.loop(0, C)
            def _pr(r):
                for j in range(C // 16):
                    a = sfr[r, pl.ds(j * 16, 16)] + dfr[r, pl.ds(j * 16, 16)]
                    pv = p1buf[pl.ds(j * 16, 16)]
                    sfr[r, pl.ds(j * 16, 16)] = jnp.where(a >= 0.0, a, pv * a)

            pltpu.sync_copy(sfr, z_h.at[pl.ds(chunk * C, C)])

    return k(sf_t, df_t, src2d, dst2d, p1)


def _sc_den(msg2d, dst2d, np_, nch):
    rpt = np_ // NS
    nbase, rem = nch // NW, nch % NW

    @functools.partial(
        pl.kernel,
        mesh=_mesh(),
        out_type=(
            jax.ShapeDtypeStruct((NC, np_), jnp.float32),
            jax.ShapeDtypeStruct((NC, np_), jnp.float32),
        ),
        scratch_types=[
            pltpu.VMEM((C,), jnp.int32),
            pltpu.VMEM((C,), jnp.float32),
            pltpu.VMEM((C,), jnp.float32),
            pltpu.MemorySpace.VMEM_SHARED((np_,), jnp.float32),
            pltpu.MemorySpace.VMEM_SHARED((np_,), jnp.float32),
        ],
    )
    def k(msg_h, dst_h, denp_h, degp_h, dstc, mbuf, onebuf, den_sh, deg_sh):
        cid = lax.axis_index("c")
        sid = lax.axis_index("s")
        wid = sid * NC + cid
        z16 = jnp.zeros((16,), jnp.float32)
        o16 = jnp.ones((16,), jnp.float32)
        for j in range(C // 16):
            mbuf[pl.ds(j * 16, 16)] = z16
            onebuf[pl.ds(j * 16, 16)] = o16
        for r in range(rpt // C):
            pltpu.sync_copy(mbuf, den_sh.at[pl.ds(sid * rpt + r * C, C)])
            pltpu.sync_copy(mbuf, deg_sh.at[pl.ds(sid * rpt + r * C, C)])
        plsc.subcore_barrier()

        nch_w = nbase + jnp.where(wid < rem, 1, 0)

        @pl.loop(0, nch_w)
        def _chunk(i):
            chunk = wid + i * NW
            pltpu.sync_copy(dst_h.at[chunk], dstc)
            pltpu.sync_copy(msg_h.at[chunk], mbuf)
            pltpu.sync_copy(mbuf, den_sh.at[dstc], add=True)
            pltpu.sync_copy(onebuf, deg_sh.at[dstc], add=True)

        plsc.subcore_barrier()
        pltpu.sync_copy(den_sh.at[pl.ds(sid * rpt, rpt)],
                        denp_h.at[cid, pl.ds(sid * rpt, rpt)])
        pltpu.sync_copy(deg_sh.at[pl.ds(sid * rpt, rpt)],
                        degp_h.at[cid, pl.ds(sid * rpt, rpt)])

    return k(msg2d, dst2d)


def kernel(x, edge_index, raw_affine, W0, a_src0, a_dst0, W1, a_src1, a_dst1,
           W2, a_src2, a_dst2, W3, a_src3, a_dst3, src_W, src_b, dst_W, dst_b,
           p1, c1_W, c1_b, p2, c2_W, c2_b):
    N, D = x.shape
    E = edge_index.shape[1]
    NP = ((N + 2047) // 2048) * 2048
    NCH = E // C

    xp = jnp.pad(x, ((0, NP - N), (0, 0)))
    src2d = edge_index[0].reshape(NCH, C)
    dst2d = edge_index[1].reshape(NCH, C)

    Ws = (W0, W1, W2, W3)
    A2s = (jnp.stack([a_src0, a_dst0], 1), jnp.stack([a_src1, a_dst1], 1),
           jnp.stack([a_src2, a_dst2], 1), jnp.stack([a_src3, a_dst3], 1))
    dins = (D, 256, 256, 128)
    douts = (256, 256, 128, 128)
    modes = ("first", "fsplit", "fsplit", "esplit")

    parts = xp
    s2d = jnp.zeros((NP, 1), jnp.float32)
    for i in range(4):
        hw, sv = _tc_stage(modes[i], dins[i], douts[i], parts, s2d, Ws[i], A2s[i], NP)
        ssrc = sv[:, 0]
        sdst = sv[:, 1]
        w2d, spart = _sc_w(ssrc, sdst, src2d, dst2d, NP, NCH)
        if douts[i] == 256:
            table = hw.reshape(2 * NP, C)
            acc = _sc_agg(table, w2d, src2d, dst2d, True, NP, NCH)
        else:
            acc = _sc_agg(hw, w2d, src2d, dst2d, False, NP, NCH)
        parts = acc
        s2d = (spart[0] + spart[1]).reshape(NP, 1)

    sf_t, df_t = _tc_sfdf(parts, s2d, src_W, src_b.reshape(1, 128),
                          dst_W, dst_b.reshape(1, 128), NP)
    z = _sc_z(sf_t, df_t, src2d, dst2d, p1, E, NCH)
    pc, msg = _tc_mlp(z, c1_W, c1_b.reshape(1, 128), p2.reshape(1, 128),
                      c2_W, c2_b.reshape(1, 2), raw_affine.reshape(E, 1), E)
    denp, degp = _sc_den(msg.reshape(NCH, C), dst2d, NP, NCH)
    pd = _tc_den(denp, degp, NP)
    return (pc, pd.reshape(NP)[:N])


# speedup vs baseline: 11.9973x; 1.0360x over previous
import functools

import jax
import jax.numpy as jnp
from jax import lax
from jax.experimental import pallas as pl
from jax.experimental.pallas import tpu as pltpu, tpu_sc as plsc

NC, NS = 2, 16
NW = NC * NS
C = 128
_EPS = 1e-9


def _mesh():
    return plsc.VectorSubcoreMesh(core_axis_name="c", subcore_axis_name="s")


def _tc_stage(mode, din, dout, parts, s2d, W, A2, np_):
    bm = 1024
    grid = (np_ // bm,)

    def body(p_ref, s_ref, W_ref, A2_ref, hw_ref, sv_ref):
        if mode == "first":
            h0 = p_ref[...]
            hw = jnp.dot(h0, W_ref[...], preferred_element_type=jnp.float32)
        elif mode == "fsplit":
            s = s_ref[...] + _EPS
            h0 = p_ref[0] / s
            h1 = p_ref[1] / s
            h0 = jnp.where(h0 > 0, h0, jnp.exp(h0) - 1.0)
            h1 = jnp.where(h1 > 0, h1, jnp.exp(h1) - 1.0)
            hw = jnp.dot(h0, W_ref[0:128, :], preferred_element_type=jnp.float32)
            hw += jnp.dot(h1, W_ref[128:256, :], preferred_element_type=jnp.float32)
        else:
            s = s_ref[...] + _EPS
            h0 = (p_ref[0] + p_ref[1]) / s
            h0 = jnp.where(h0 > 0, h0, jnp.exp(h0) - 1.0)
            hw = jnp.dot(h0, W_ref[...], preferred_element_type=jnp.float32)
        sv_ref[...] = jnp.dot(hw, A2_ref[...], preferred_element_type=jnp.float32)
        if dout == 256:
            hw_ref[0] = hw[:, 0:128]
            hw_ref[1] = hw[:, 128:256]
        else:
            hw_ref[...] = hw

    if mode == "first":
        p_spec = pl.BlockSpec((bm, 128), lambda m: (m, 0))
    else:
        p_spec = pl.BlockSpec((2, bm, 128), lambda m: (0, m, 0))
    if dout == 256:
        hw_shape = jax.ShapeDtypeStruct((2, np_, 128), jnp.float32)
        hw_spec = pl.BlockSpec((2, bm, 128), lambda m: (0, m, 0))
    else:
        hw_shape = jax.ShapeDtypeStruct((np_, 128), jnp.float32)
        hw_spec = pl.BlockSpec((bm, 128), lambda m: (m, 0))

    return pl.pallas_call(
        body,
        grid=grid,
        in_specs=[
            p_spec,
            pl.BlockSpec((bm, 1), lambda m: (m, 0)),
            pl.BlockSpec((din, dout), lambda m: (0, 0)),
            pl.BlockSpec((dout, 2), lambda m: (0, 0)),
        ],
        out_specs=[hw_spec, pl.BlockSpec((bm, 2), lambda m: (m, 0))],
        out_shape=[hw_shape, jax.ShapeDtypeStruct((np_, 2), jnp.float32)],
    )(parts, s2d, W, A2)


def _tc_sfdf(acc, s2d, srcW, srcb, dstW, dstb, np_):
    bm = 1024
    grid = (np_ // bm,)

    def body(p_ref, s_ref, sw_ref, sb_ref, dw_ref, db_ref, sf_ref, df_ref):
        s = s_ref[...] + _EPS
        h = (p_ref[0] + p_ref[1]) / s
        h = jnp.where(h > 0, h, jnp.exp(h) - 1.0)
        sf_ref[...] = jnp.dot(h, sw_ref[...], preferred_element_type=jnp.float32) + sb_ref[...]
        df_ref[...] = jnp.dot(h, dw_ref[...], preferred_element_type=jnp.float32) + db_ref[...]

    io = pl.BlockSpec((bm, 128), lambda m: (m, 0))
    wspec = pl.BlockSpec((128, 128), lambda m: (0, 0))
    bspec = pl.BlockSpec((1, 128), lambda m: (0, 0))
    return pl.pallas_call(
        body,
        grid=grid,
        in_specs=[pl.BlockSpec((2, bm, 128), lambda m: (0, m, 0)),
                  pl.BlockSpec((bm, 1), lambda m: (m, 0)), wspec, bspec, wspec, bspec],
        out_specs=[io, io],
        out_shape=[jax.ShapeDtypeStruct((np_, 128), jnp.float32)] * 2,
    )(acc, s2d, srcW, srcb, dstW, dstb)


def _tc_mlp(z, c1W, c1b, p2, c2W, c2b, ra, e_):
    bm = 640
    grid = (e_ // bm,)

    def body(z_ref, w1_ref, b1_ref, p2_ref, w2_ref, b2_ref, ra_ref, pc_ref, msg_ref):
        u = jnp.dot(z_ref[...], w1_ref[...], preferred_element_type=jnp.float32) + b1_ref[...]
        u = jnp.where(u >= 0, u, p2_ref[...] * u)
        pc = jnp.dot(u, w2_ref[...], preferred_element_type=jnp.float32) + b2_ref[...]
        pc_ref[...] = pc
        p0 = pc[:, 0:1]
        p1c = pc[:, 1:2]
        m = jnp.maximum(p0, p1c)
        e0 = jnp.exp(p0 - m)
        e1 = jnp.exp(p1c - m)
        msg_ref[...] = ra_ref[...] * (e1 - e0) / (e0 + e1)

    return pl.pallas_call(
        body,
        grid=grid,
        in_specs=[
            pl.BlockSpec((bm, 128), lambda m: (m, 0)),
            pl.BlockSpec((128, 128), lambda m: (0, 0)),
            pl.BlockSpec((1, 128), lambda m: (0, 0)),
            pl.BlockSpec((1, 128), lambda m: (0, 0)),
            pl.BlockSpec((128, 2), lambda m: (0, 0)),
            pl.BlockSpec((1, 2), lambda m: (0, 0)),
            pl.BlockSpec((bm, 1), lambda m: (m, 0)),
        ],
        out_specs=[pl.BlockSpec((bm, 2), lambda m: (m, 0)),
                   pl.BlockSpec((bm, 1), lambda m: (m, 0))],
        out_shape=[jax.ShapeDtypeStruct((e_, 2), jnp.float32),
                   jax.ShapeDtypeStruct((e_, 1), jnp.float32)],
    )(z, c1W, c1b, p2, c2W, c2b, ra)


def _tc_den(denp, degp, np_):
    def body(den_ref, deg_ref, o_ref):
        d = den_ref[0:1, :] + den_ref[1:2, :]
        g = deg_ref[0:1, :] + deg_ref[1:2, :]
        o_ref[...] = d / jnp.maximum(g, 1.0)

    return pl.pallas_call(
        body,
        out_shape=jax.ShapeDtypeStruct((1, np_), jnp.float32),
    )(denp, degp)


def _sc_layer(ssrc_rep, sdst, table, src2d, dst2d, fsplit, np_, nch):
    rpt = np_ // NS
    if fsplit:
        total = nch // NS
    else:
        total = nch // (NC * NS)
    half = total // 2

    @functools.partial(
        pl.kernel,
        mesh=_mesh(),
        out_type=(
            jax.ShapeDtypeStruct((NC, np_, C), jnp.float32),
            jax.ShapeDtypeStruct((NC, np_), jnp.float32),
        ),
        scratch_types=[
            pltpu.VMEM((C,), jnp.int32),
            pltpu.VMEM((C,), jnp.int32),
            pltpu.VMEM((C,), jnp.int32),
            pltpu.VMEM((C,), jnp.int32),
            pltpu.VMEM((C, C), jnp.float32),
            pltpu.VMEM((C, C), jnp.float32),
            pltpu.VMEM((C,), jnp.float32),
            pltpu.VMEM((C,), jnp.float32),
            pltpu.VMEM((C,), jnp.float32),
            pltpu.VMEM((C,), jnp.float32),
            pltpu.VMEM((C,), jnp.float32),
            pltpu.VMEM((C,), jnp.float32),
            pltpu.MemorySpace.VMEM_SHARED((np_, C), jnp.float32),
            pltpu.MemorySpace.VMEM_SHARED((np_,), jnp.float32),
            pltpu.SemaphoreType.DMA,
            pltpu.SemaphoreType.DMA,
            pltpu.SemaphoreType.DMA,
            pltpu.SemaphoreType.DMA,
            pltpu.SemaphoreType.DMA,
            pltpu.SemaphoreType.DMA,
        ],
    )
    def k(ssrc_h, sdst_h, tbl_h, src_h, dst_h, acc_h, spart_h,
          srcc0, srcc1, dstc0, dstc1, rows0, rows1, vs0, vs1, vd0, vd1, wb0, wb1,
          acc_sh, s_sh, semr0, semr1, semv0, semv1, semd0, semd1):
        cid = lax.axis_index("c")
        sid = lax.axis_index("s")
        z16 = jnp.zeros((16,), jnp.float32)

        @pl.loop(0, C)
        def _zr(r):
            for j in range(C // 16):
                rows0[r, pl.ds(j * 16, 16)] = z16

        for j in range(C // 16):
            wb0[pl.ds(j * 16, 16)] = z16
        for b in range(rpt // C):
            pltpu.sync_copy(rows0, acc_sh.at[pl.ds(sid * rpt + b * C, C)])
            pltpu.sync_copy(wb0, s_sh.at[pl.ds(sid * rpt + b * C, C)])
        plsc.subcore_barrier()

        if fsplit:
            first = sid * total
        else:
            first = (cid * NS + sid) * total

        rows = (rows0, rows1)
        srcc = (srcc0, srcc1)
        dstc = (dstc0, dstc1)
        vs = (vs0, vs1)
        vd = (vd0, vd1)
        wb = (wb0, wb1)
        semr = (semr0, semr1)
        semv = (semv0, semv1)
        semd = (semd0, semd1)

        def fire(p, j):
            pltpu.sync_copy(src_h.at[first + j], srcc[p])
            pltpu.sync_copy(dst_h.at[first + j], dstc[p])
            if fsplit:
                off = (cid * np_).astype(jnp.int32)
                for b in range(C // 16):
                    srcc[p][pl.ds(b * 16, 16)] = srcc[p][pl.ds(b * 16, 16)] + off
            pltpu.async_copy(tbl_h.at[srcc[p]], rows[p], semr[p])
            pltpu.async_copy(ssrc_h.at[srcc[p]], vs[p], semv[p])
            pltpu.async_copy(sdst_h.at[dstc[p]], vd[p], semd[p])

        def process(p):
            pltpu.make_async_copy(tbl_h.at[srcc[p]], rows[p], semr[p]).wait()
            pltpu.make_async_copy(ssrc_h.at[srcc[p]], vs[p], semv[p]).wait()
            pltpu.make_async_copy(sdst_h.at[dstc[p]], vd[p], semd[p]).wait()
            for b in range(C // 16):
                e = vs[p][pl.ds(b * 16, 16)] + vd[p][pl.ds(b * 16, 16)]
                e = jnp.where(e >= 0.0, e, 0.2 * e)
                w16 = jnp.exp(e)
                wb[p][pl.ds(b * 16, 16)] = w16
                vs[p][pl.ds(b * 16, 16)] = w16 * (0.5 if fsplit else 1.0)
            pltpu.sync_copy(vs[p], s_sh.at[dstc[p]], add=True)

            @pl.loop(0, C // 16)
            def _scale(g):
                w16 = wb[p][pl.ds(g * 16, 16)]
                for l in range(16):
                    r = g * 16 + l
                    wr = w16[l]
                    for b in range(C // 16):
                        rows[p][r, pl.ds(b * 16, 16)] = rows[p][r, pl.ds(b * 16, 16)] * wr

            pltpu.sync_copy(rows[p], acc_sh.at[dstc[p]], add=True)

        fire(0, 0)
        fire(1, 1)

        @pl.loop(0, half - 1)
        def _main(i):
            j0 = 2 * i
            process(0)
            fire(0, j0 + 2)
            process(1)
            fire(1, j0 + 3)

        process(0)
        process(1)

        plsc.subcore_barrier()
        pltpu.sync_copy(
            acc_sh.at[pl.ds(sid * rpt, rpt)],
            acc_h.at[cid, pl.ds(sid * rpt, rpt)],
        )
        pltpu.sync_copy(
            s_sh.at[pl.ds(sid * rpt, rpt)],
            spart_h.at[cid, pl.ds(sid * rpt, rpt)],
        )

    return k(ssrc_rep, sdst, table, src2d, dst2d)


def _sc_z(sf_t, df_t, src2d, dst2d, p1, e_, nch):
    total = nch // NW
    half = total // 2

    @functools.partial(
        pl.kernel,
        mesh=_mesh(),
        out_type=jax.ShapeDtypeStruct((e_, C), jnp.float32),
        scratch_types=[
            pltpu.VMEM((total, C), jnp.int32),
            pltpu.VMEM((total, C), jnp.int32),
            pltpu.VMEM((C,), jnp.float32),
            pltpu.VMEM((C, C), jnp.float32),
            pltpu.VMEM((C, C), jnp.float32),
            pltpu.VMEM((C, C), jnp.float32),
            pltpu.VMEM((C, C), jnp.float32),
            pltpu.SemaphoreType.DMA,
            pltpu.SemaphoreType.DMA,
            pltpu.SemaphoreType.DMA,
            pltpu.SemaphoreType.DMA,
        ],
    )
    def k(sf_h, df_h, src_h, dst_h, p1_h, z_h, srcall, dstall, p1buf,
          sfr0, sfr1, dfr0, dfr1, sems0, sems1, semd0, semd1):
        cid = lax.axis_index("c")
        sid = lax.axis_index("s")
        wid = sid * NC + cid
        first = wid * total
        pltpu.sync_copy(p1_h, p1buf)
        pltpu.sync_copy(src_h.at[pl.ds(first, total)], srcall)
        pltpu.sync_copy(dst_h.at[pl.ds(first, total)], dstall)

        sfr = (sfr0, sfr1)
        dfr = (dfr0, dfr1)
        sems = (sems0, sems1)
        semd = (semd0, semd1)

        def fire(p, j):
            pltpu.async_copy(sf_h.at[srcall.at[j]], sfr[p], sems[p])
            pltpu.async_copy(df_h.at[dstall.at[j]], dfr[p], semd[p])

        def process(p, j):
            pltpu.make_async_copy(sf_h.at[srcall.at[j]], sfr[p], sems[p]).wait()
            pltpu.make_async_copy(df_h.at[dstall.at[j]], dfr[p], semd[p]).wait()

            @pl.loop(0, C)
            def _pr(r):
                for b in range(C // 16):
                    a = sfr[p][r, pl.ds(b * 16, 16)] + dfr[p][r, pl.ds(b * 16, 16)]
                    pv = p1buf[pl.ds(b * 16, 16)]
                    sfr[p][r, pl.ds(b * 16, 16)] = jnp.where(a >= 0.0, a, pv * a)

            pltpu.sync_copy(sfr[p], z_h.at[pl.ds((first + j) * C, C)])

        fire(0, 0)
        fire(1, 1)

        @pl.loop(0, half - 1)
        def _main(i):
            j0 = 2 * i
            process(0, j0)
            fire(0, j0 + 2)
            process(1, j0 + 1)
            fire(1, j0 + 3)

        process(0, total - 2)
        process(1, total - 1)

    return k(sf_t, df_t, src2d, dst2d, p1)


def _sc_den(msg2d, dst2d, np_, nch):
    rpt = np_ // NS
    total = nch // NW

    @functools.partial(
        pl.kernel,
        mesh=_mesh(),
        out_type=(
            jax.ShapeDtypeStruct((NC, np_), jnp.float32),
            jax.ShapeDtypeStruct((NC, np_), jnp.float32),
        ),
        scratch_types=[
            pltpu.VMEM((C,), jnp.int32),
            pltpu.VMEM((C,), jnp.float32),
            pltpu.VMEM((C,), jnp.float32),
            pltpu.MemorySpace.VMEM_SHARED((np_,), jnp.float32),
            pltpu.MemorySpace.VMEM_SHARED((np_,), jnp.float32),
        ],
    )
    def k(msg_h, dst_h, denp_h, degp_h, dstc, mbuf, onebuf, den_sh, deg_sh):
        cid = lax.axis_index("c")
        sid = lax.axis_index("s")
        wid = sid * NC + cid
        z16 = jnp.zeros((16,), jnp.float32)
        o16 = jnp.ones((16,), jnp.float32)
        for j in range(C // 16):
            mbuf[pl.ds(j * 16, 16)] = z16
            onebuf[pl.ds(j * 16, 16)] = o16
        for r in range(rpt // C):
            pltpu.sync_copy(mbuf, den_sh.at[pl.ds(sid * rpt + r * C, C)])
            pltpu.sync_copy(mbuf, deg_sh.at[pl.ds(sid * rpt + r * C, C)])
        plsc.subcore_barrier()

        @pl.loop(0, total)
        def _chunk(i):
            chunk = wid * total + i
            pltpu.sync_copy(dst_h.at[chunk], dstc)
            pltpu.sync_copy(msg_h.at[chunk], mbuf)
            pltpu.sync_copy(mbuf, den_sh.at[dstc], add=True)
            pltpu.sync_copy(onebuf, deg_sh.at[dstc], add=True)

        plsc.subcore_barrier()
        pltpu.sync_copy(den_sh.at[pl.ds(sid * rpt, rpt)],
                        denp_h.at[cid, pl.ds(sid * rpt, rpt)])
        pltpu.sync_copy(deg_sh.at[pl.ds(sid * rpt, rpt)],
                        degp_h.at[cid, pl.ds(sid * rpt, rpt)])

    return k(msg2d, dst2d)


def kernel(x, edge_index, raw_affine, W0, a_src0, a_dst0, W1, a_src1, a_dst1,
           W2, a_src2, a_dst2, W3, a_src3, a_dst3, src_W, src_b, dst_W, dst_b,
           p1, c1_W, c1_b, p2, c2_W, c2_b):
    N, D = x.shape
    E = edge_index.shape[1]
    NP = ((N + 2047) // 2048) * 2048
    NCH = -(-E // (256 * C)) * 256
    EP = NCH * C

    xp = jnp.pad(x, ((0, NP - N), (0, 0)))
    pad_idx = jnp.full((EP - E,), NP - 1, jnp.int32)
    src2d = jnp.concatenate([edge_index[0], pad_idx]).reshape(NCH, C)
    dst2d = jnp.concatenate([edge_index[1], pad_idx]).reshape(NCH, C)
    rap = jnp.pad(raw_affine, (0, EP - E)).reshape(EP, 1)

    Ws = (W0, W1, W2, W3)
    A2s = (jnp.stack([a_src0, a_dst0], 1), jnp.stack([a_src1, a_dst1], 1),
           jnp.stack([a_src2, a_dst2], 1), jnp.stack([a_src3, a_dst3], 1))
    dins = (D, 256, 256, 128)
    douts = (256, 256, 128, 128)
    modes = ("first", "fsplit", "fsplit", "esplit")

    parts = xp
    s2d = jnp.zeros((NP, 1), jnp.float32)
    for i in range(4):
        hw, sv = _tc_stage(modes[i], dins[i], douts[i], parts, s2d, Ws[i], A2s[i], NP)
        ssrc = sv[:, 0]
        sdst = sv[:, 1]
        if douts[i] == 256:
            table = hw.reshape(2 * NP, C)
            ssrc_rep = jnp.concatenate([ssrc, ssrc])
            acc, spart = _sc_layer(ssrc_rep, sdst, table, src2d, dst2d, True, NP, NCH)
        else:
            acc, spart = _sc_layer(ssrc, sdst, hw, src2d, dst2d, False, NP, NCH)
        parts = acc
        s2d = (spart[0] + spart[1]).reshape(NP, 1)

    sf_t, df_t = _tc_sfdf(parts, s2d, src_W, src_b.reshape(1, 128),
                          dst_W, dst_b.reshape(1, 128), NP)
    z = _sc_z(sf_t, df_t, src2d, dst2d, p1, EP, NCH)
    pc, msg = _tc_mlp(z, c1_W, c1_b.reshape(1, 128), p2.reshape(1, 128),
                      c2_W, c2_b.reshape(1, 2), rap, EP)
    denp, degp = _sc_den(msg.reshape(NCH, C), dst2d, NP, NCH)
    pd = _tc_den(denp, degp, NP)
    return (pc[:E], pd.reshape(NP)[:N])


# speedup vs baseline: 12.2836x; 1.0239x over previous
import functools

import jax
import jax.numpy as jnp
from jax import lax
from jax.experimental import pallas as pl
from jax.experimental.pallas import tpu as pltpu, tpu_sc as plsc

NC, NS = 2, 16
NW = NC * NS
C = 128
_EPS = 1e-9


def _mesh():
    return plsc.VectorSubcoreMesh(core_axis_name="c", subcore_axis_name="s")


def _tc_stage(mode, din, dout, parts, s2d, W, A2, np_):
    bm = 1024
    grid = (np_ // bm,)

    def body(p_ref, s_ref, W_ref, A2_ref, hw_ref, sv_ref):
        if mode == "first":
            h0 = p_ref[...]
            hw = jnp.dot(h0, W_ref[...], preferred_element_type=jnp.float32)
        elif mode == "fsplit":
            s = s_ref[...] + _EPS
            h0 = p_ref[0] / s
            h1 = p_ref[1] / s
            h0 = jnp.where(h0 > 0, h0, jnp.exp(h0) - 1.0)
            h1 = jnp.where(h1 > 0, h1, jnp.exp(h1) - 1.0)
            hw = jnp.dot(h0, W_ref[0:128, :], preferred_element_type=jnp.float32)
            hw += jnp.dot(h1, W_ref[128:256, :], preferred_element_type=jnp.float32)
        else:
            s = s_ref[...] + _EPS
            h0 = (p_ref[0] + p_ref[1]) / s
            h0 = jnp.where(h0 > 0, h0, jnp.exp(h0) - 1.0)
            hw = jnp.dot(h0, W_ref[...], preferred_element_type=jnp.float32)
        sv_ref[...] = jnp.dot(hw, A2_ref[...], preferred_element_type=jnp.float32)
        if dout == 256:
            hw_ref[0] = hw[:, 0:128]
            hw_ref[1] = hw[:, 128:256]
        else:
            hw_ref[...] = hw

    if mode == "first":
        p_spec = pl.BlockSpec((bm, 128), lambda m: (m, 0))
    else:
        p_spec = pl.BlockSpec((2, bm, 128), lambda m: (0, m, 0))
    if dout == 256:
        hw_shape = jax.ShapeDtypeStruct((2, np_, 128), jnp.float32)
        hw_spec = pl.BlockSpec((2, bm, 128), lambda m: (0, m, 0))
    else:
        hw_shape = jax.ShapeDtypeStruct((np_, 128), jnp.float32)
        hw_spec = pl.BlockSpec((bm, 128), lambda m: (m, 0))

    return pl.pallas_call(
        body,
        grid=grid,
        in_specs=[
            p_spec,
            pl.BlockSpec((bm, 1), lambda m: (m, 0)),
            pl.BlockSpec((din, dout), lambda m: (0, 0)),
            pl.BlockSpec((dout, 2), lambda m: (0, 0)),
        ],
        out_specs=[hw_spec, pl.BlockSpec((bm, 2), lambda m: (m, 0))],
        out_shape=[hw_shape, jax.ShapeDtypeStruct((np_, 2), jnp.float32)],
    )(parts, s2d, W, A2)


def _tc_sfdf(acc, s2d, srcW, srcb, dstW, dstb, np_):
    bm = 1024
    grid = (np_ // bm,)

    def body(p_ref, s_ref, sw_ref, sb_ref, dw_ref, db_ref, sf_ref, df_ref):
        s = s_ref[...] + _EPS
        h = (p_ref[0] + p_ref[1]) / s
        h = jnp.where(h > 0, h, jnp.exp(h) - 1.0)
        sf_ref[...] = jnp.dot(h, sw_ref[...], preferred_element_type=jnp.float32) + sb_ref[...]
        df_ref[...] = jnp.dot(h, dw_ref[...], preferred_element_type=jnp.float32) + db_ref[...]

    io = pl.BlockSpec((bm, 128), lambda m: (m, 0))
    wspec = pl.BlockSpec((128, 128), lambda m: (0, 0))
    bspec = pl.BlockSpec((1, 128), lambda m: (0, 0))
    return pl.pallas_call(
        body,
        grid=grid,
        in_specs=[pl.BlockSpec((2, bm, 128), lambda m: (0, m, 0)),
                  pl.BlockSpec((bm, 1), lambda m: (m, 0)), wspec, bspec, wspec, bspec],
        out_specs=[io, io],
        out_shape=[jax.ShapeDtypeStruct((np_, 128), jnp.float32)] * 2,
    )(acc, s2d, srcW, srcb, dstW, dstb)


def _tc_mlp(z, c1W, c1b, p2, c2W, c2b, ra, e_):
    bm = 640
    grid = (e_ // bm,)

    def body(z_ref, w1_ref, b1_ref, p2_ref, w2_ref, b2_ref, ra_ref, pc_ref, msg_ref):
        u = jnp.dot(z_ref[...], w1_ref[...], preferred_element_type=jnp.float32) + b1_ref[...]
        u = jnp.where(u >= 0, u, p2_ref[...] * u)
        pc = jnp.dot(u, w2_ref[...], preferred_element_type=jnp.float32) + b2_ref[...]
        pc_ref[...] = pc
        p0 = pc[:, 0:1]
        p1c = pc[:, 1:2]
        m = jnp.maximum(p0, p1c)
        e0 = jnp.exp(p0 - m)
        e1 = jnp.exp(p1c - m)
        msg_ref[...] = ra_ref[...] * (e1 - e0) / (e0 + e1)

    return pl.pallas_call(
        body,
        grid=grid,
        in_specs=[
            pl.BlockSpec((bm, 128), lambda m: (m, 0)),
            pl.BlockSpec((128, 128), lambda m: (0, 0)),
            pl.BlockSpec((1, 128), lambda m: (0, 0)),
            pl.BlockSpec((1, 128), lambda m: (0, 0)),
            pl.BlockSpec((128, 2), lambda m: (0, 0)),
            pl.BlockSpec((1, 2), lambda m: (0, 0)),
            pl.BlockSpec((bm, 1), lambda m: (m, 0)),
        ],
        out_specs=[pl.BlockSpec((bm, 2), lambda m: (m, 0)),
                   pl.BlockSpec((bm, 1), lambda m: (m, 0))],
        out_shape=[jax.ShapeDtypeStruct((e_, 2), jnp.float32),
                   jax.ShapeDtypeStruct((e_, 1), jnp.float32)],
    )(z, c1W, c1b, p2, c2W, c2b, ra)


def _tc_den(denp, degp, np_):
    def body(den_ref, deg_ref, o_ref):
        d = den_ref[0:1, :] + den_ref[1:2, :]
        g = deg_ref[0:1, :] + deg_ref[1:2, :]
        o_ref[...] = d / jnp.maximum(g, 1.0)

    return pl.pallas_call(
        body,
        out_shape=jax.ShapeDtypeStruct((1, np_), jnp.float32),
    )(denp, degp)


def _sc_layer(ssrc_rep, sdst, table, src2d, dst2d, fsplit, np_, nch):
    rpt = np_ // NS
    if fsplit:
        total = nch // NS
    else:
        total = nch // (NC * NS)
    half = total // 2

    @functools.partial(
        pl.kernel,
        mesh=_mesh(),
        out_type=(
            jax.ShapeDtypeStruct((NC, np_, C), jnp.float32),
            jax.ShapeDtypeStruct((NC, np_), jnp.float32),
        ),
        scratch_types=[
            pltpu.VMEM((8, C), jnp.int32),
            pltpu.VMEM((8, C), jnp.int32),
            pltpu.VMEM((8, C), jnp.int32),
            pltpu.VMEM((8, C), jnp.int32),
            pltpu.VMEM((C,), jnp.int32),
            pltpu.VMEM((C,), jnp.int32),
            pltpu.VMEM((C, C), jnp.float32),
            pltpu.VMEM((C, C), jnp.float32),
            pltpu.VMEM((C,), jnp.float32),
            pltpu.VMEM((C,), jnp.float32),
            pltpu.VMEM((C,), jnp.float32),
            pltpu.VMEM((C,), jnp.float32),
            pltpu.VMEM((C,), jnp.float32),
            pltpu.VMEM((C,), jnp.float32),
            pltpu.MemorySpace.VMEM_SHARED((np_, C), jnp.float32),
            pltpu.MemorySpace.VMEM_SHARED((np_,), jnp.float32),
            pltpu.SemaphoreType.DMA,
            pltpu.SemaphoreType.DMA,
            pltpu.SemaphoreType.DMA,
            pltpu.SemaphoreType.DMA,
            pltpu.SemaphoreType.DMA,
            pltpu.SemaphoreType.DMA,
        ],
    )
    def k(ssrc_h, sdst_h, tbl_h, src_h, dst_h, acc_h, spart_h,
          srcb0, srcb1, dstb0, dstb1, dtmp0, dtmp1, rows0, rows1,
          vs0, vs1, vd0, vd1, wb0, wb1,
          acc_sh, s_sh, semr0, semr1, semv0, semv1, semd0, semd1):
        cid = lax.axis_index("c")
        sid = lax.axis_index("s")
        z16 = jnp.zeros((16,), jnp.float32)

        @pl.loop(0, C)
        def _zr(r):
            for j in range(C // 16):
                rows0[r, pl.ds(j * 16, 16)] = z16

        for j in range(C // 16):
            wb0[pl.ds(j * 16, 16)] = z16
        for b in range(rpt // C):
            pltpu.sync_copy(rows0, acc_sh.at[pl.ds(sid * rpt + b * C, C)])
            pltpu.sync_copy(wb0, s_sh.at[pl.ds(sid * rpt + b * C, C)])
        plsc.subcore_barrier()

        if fsplit:
            first = sid * total
        else:
            first = (cid * NS + sid) * total
        ngr = total // 8

        rows = (rows0, rows1)
        srcb = (srcb0, srcb1)
        dstb = (dstb0, dstb1)
        dtmp = (dtmp0, dtmp1)
        vs = (vs0, vs1)
        vd = (vd0, vd1)
        wb = (wb0, wb1)
        semr = (semr0, semr1)
        semv = (semv0, semv1)
        semd = (semd0, semd1)

        def load_group(q, gp):
            pltpu.sync_copy(src_h.at[pl.ds(first + q * 8, 8)], srcb[gp])
            pltpu.sync_copy(dst_h.at[pl.ds(first + q * 8, 8)], dstb[gp])
            if fsplit:
                off = (cid * np_).astype(jnp.int32)
                for jj in range(8):
                    for b in range(C // 16):
                        srcb[gp][jj, pl.ds(b * 16, 16)] = srcb[gp][jj, pl.ds(b * 16, 16)] + off

        def fire(p, gp, jj):
            pltpu.async_copy(tbl_h.at[srcb[gp].at[jj]], rows[p], semr[p])
            pltpu.async_copy(ssrc_h.at[srcb[gp].at[jj]], vs[p], semv[p])
            pltpu.async_copy(sdst_h.at[dstb[gp].at[jj]], vd[p], semd[p])

        def process(p, gp, jj):
            pltpu.make_async_copy(tbl_h.at[srcb[gp].at[jj]], rows[p], semr[p]).wait()
            pltpu.make_async_copy(ssrc_h.at[srcb[gp].at[jj]], vs[p], semv[p]).wait()
            pltpu.make_async_copy(sdst_h.at[dstb[gp].at[jj]], vd[p], semd[p]).wait()
            for b in range(C // 16):
                e = vs[p][pl.ds(b * 16, 16)] + vd[p][pl.ds(b * 16, 16)]
                e = jnp.where(e >= 0.0, e, 0.2 * e)
                w16 = jnp.exp(e)
                wb[p][pl.ds(b * 16, 16)] = w16
                vs[p][pl.ds(b * 16, 16)] = w16 * (0.5 if fsplit else 1.0)
                dtmp[p][pl.ds(b * 16, 16)] = dstb[gp][jj, pl.ds(b * 16, 16)]
            pltpu.sync_copy(vs[p], s_sh.at[dtmp[p]], add=True)

            @pl.loop(0, C // 16)
            def _scale(g):
                w16 = wb[p][pl.ds(g * 16, 16)]
                for l in range(16):
                    r = g * 16 + l
                    wr = w16[l]
                    for b in range(C // 16):
                        rows[p][r, pl.ds(b * 16, 16)] = rows[p][r, pl.ds(b * 16, 16)] * wr

            pltpu.sync_copy(rows[p], acc_sh.at[dtmp[p]], add=True)

        def group_body(q, gp, load_next, last):
            if load_next:
                load_group(q + 1, 1 - gp)

            @pl.loop(0, 3)
            def _pairs(i):
                jj0 = 2 * i
                process(0, gp, jj0)
                fire(0, gp, jj0 + 2)
                process(1, gp, jj0 + 1)
                fire(1, gp, jj0 + 3)

            process(0, gp, 6)
            if not last:
                fire(0, 1 - gp, 0)
            process(1, gp, 7)
            if not last:
                fire(1, 1 - gp, 1)

        load_group(0, 0)
        fire(0, 0, 0)
        fire(1, 0, 1)

        @pl.loop(0, ngr // 2 - 1)
        def _main(t):
            q0 = 2 * t
            group_body(q0, 0, True, False)
            group_body(q0 + 1, 1, True, False)

        group_body(ngr - 2, 0, True, False)
        group_body(ngr - 1, 1, False, True)

        plsc.subcore_barrier()
        pltpu.sync_copy(
            acc_sh.at[pl.ds(sid * rpt, rpt)],
            acc_h.at[cid, pl.ds(sid * rpt, rpt)],
        )
        pltpu.sync_copy(
            s_sh.at[pl.ds(sid * rpt, rpt)],
            spart_h.at[cid, pl.ds(sid * rpt, rpt)],
        )

    return k(ssrc_rep, sdst, table, src2d, dst2d)


def _sc_z(sf_t, df_t, src2d, dst2d, p1, e_, nch):
    total = nch // NW
    half = total // 2

    @functools.partial(
        pl.kernel,
        mesh=_mesh(),
        out_type=jax.ShapeDtypeStruct((e_, C), jnp.float32),
        scratch_types=[
            pltpu.VMEM((total, C), jnp.int32),
            pltpu.VMEM((total, C), jnp.int32),
            pltpu.VMEM((C,), jnp.float32),
            pltpu.VMEM((C, C), jnp.float32),
            pltpu.VMEM((C, C), jnp.float32),
            pltpu.VMEM((C, C), jnp.float32),
            pltpu.VMEM((C, C), jnp.float32),
            pltpu.SemaphoreType.DMA,
            pltpu.SemaphoreType.DMA,
            pltpu.SemaphoreType.DMA,
            pltpu.SemaphoreType.DMA,
        ],
    )
    def k(sf_h, df_h, src_h, dst_h, p1_h, z_h, srcall, dstall, p1buf,
          sfr0, sfr1, dfr0, dfr1, sems0, sems1, semd0, semd1):
        cid = lax.axis_index("c")
        sid = lax.axis_index("s")
        wid = sid * NC + cid
        first = wid * total
        pltpu.sync_copy(p1_h, p1buf)
        pltpu.sync_copy(src_h.at[pl.ds(first, total)], srcall)
        pltpu.sync_copy(dst_h.at[pl.ds(first, total)], dstall)

        sfr = (sfr0, sfr1)
        dfr = (dfr0, dfr1)
        sems = (sems0, sems1)
        semd = (semd0, semd1)

        def fire(p, j):
            pltpu.async_copy(sf_h.at[srcall.at[j]], sfr[p], sems[p])
            pltpu.async_copy(df_h.at[dstall.at[j]], dfr[p], semd[p])

        def process(p, j):
            pltpu.make_async_copy(sf_h.at[srcall.at[j]], sfr[p], sems[p]).wait()
            pltpu.make_async_copy(df_h.at[dstall.at[j]], dfr[p], semd[p]).wait()

            @pl.loop(0, C)
            def _pr(r):
                for b in range(C // 16):
                    a = sfr[p][r, pl.ds(b * 16, 16)] + dfr[p][r, pl.ds(b * 16, 16)]
                    pv = p1buf[pl.ds(b * 16, 16)]
                    sfr[p][r, pl.ds(b * 16, 16)] = jnp.where(a >= 0.0, a, pv * a)

            pltpu.sync_copy(sfr[p], z_h.at[pl.ds((first + j) * C, C)])

        fire(0, 0)
        fire(1, 1)

        @pl.loop(0, half - 1)
        def _main(i):
            j0 = 2 * i
            process(0, j0)
            fire(0, j0 + 2)
            process(1, j0 + 1)
            fire(1, j0 + 3)

        process(0, total - 2)
        process(1, total - 1)

    return k(sf_t, df_t, src2d, dst2d, p1)


def _sc_den(msg2d, dst2d, np_, nch):
    rpt = np_ // NS
    total = nch // NW

    @functools.partial(
        pl.kernel,
        mesh=_mesh(),
        out_type=(
            jax.ShapeDtypeStruct((NC, np_), jnp.float32),
            jax.ShapeDtypeStruct((NC, np_), jnp.float32),
        ),
        scratch_types=[
            pltpu.VMEM((C,), jnp.int32),
            pltpu.VMEM((C,), jnp.float32),
            pltpu.VMEM((C,), jnp.float32),
            pltpu.MemorySpace.VMEM_SHARED((np_,), jnp.float32),
            pltpu.MemorySpace.VMEM_SHARED((np_,), jnp.float32),
        ],
    )
    def k(msg_h, dst_h, denp_h, degp_h, dstc, mbuf, onebuf, den_sh, deg_sh):
        cid = lax.axis_index("c")
        sid = lax.axis_index("s")
        wid = sid * NC + cid
        z16 = jnp.zeros((16,), jnp.float32)
        o16 = jnp.ones((16,), jnp.float32)
        for j in range(C // 16):
            mbuf[pl.ds(j * 16, 16)] = z16
            onebuf[pl.ds(j * 16, 16)] = o16
        for r in range(rpt // C):
            pltpu.sync_copy(mbuf, den_sh.at[pl.ds(sid * rpt + r * C, C)])
            pltpu.sync_copy(mbuf, deg_sh.at[pl.ds(sid * rpt + r * C, C)])
        plsc.subcore_barrier()

        @pl.loop(0, total)
        def _chunk(i):
            chunk = wid * total + i
            pltpu.sync_copy(dst_h.at[chunk], dstc)
            pltpu.sync_copy(msg_h.at[chunk], mbuf)
            pltpu.sync_copy(mbuf, den_sh.at[dstc], add=True)
            pltpu.sync_copy(onebuf, deg_sh.at[dstc], add=True)

        plsc.subcore_barrier()
        pltpu.sync_copy(den_sh.at[pl.ds(sid * rpt, rpt)],
                        denp_h.at[cid, pl.ds(sid * rpt, rpt)])
        pltpu.sync_copy(deg_sh.at[pl.ds(sid * rpt, rpt)],
                        degp_h.at[cid, pl.ds(sid * rpt, rpt)])

    return k(msg2d, dst2d)


def kernel(x, edge_index, raw_affine, W0, a_src0, a_dst0, W1, a_src1, a_dst1,
           W2, a_src2, a_dst2, W3, a_src3, a_dst3, src_W, src_b, dst_W, dst_b,
           p1, c1_W, c1_b, p2, c2_W, c2_b):
    N, D = x.shape
    E = edge_index.shape[1]
    NP = ((N + 2047) // 2048) * 2048
    NCH = -(-E // (256 * C)) * 256
    EP = NCH * C

    xp = jnp.pad(x, ((0, NP - N), (0, 0)))
    pad_idx = jnp.full((EP - E,), NP - 1, jnp.int32)
    src2d = jnp.concatenate([edge_index[0], pad_idx]).reshape(NCH, C)
    dst2d = jnp.concatenate([edge_index[1], pad_idx]).reshape(NCH, C)
    rap = jnp.pad(raw_affine, (0, EP - E)).reshape(EP, 1)

    Ws = (W0, W1, W2, W3)
    A2s = (jnp.stack([a_src0, a_dst0], 1), jnp.stack([a_src1, a_dst1], 1),
           jnp.stack([a_src2, a_dst2], 1), jnp.stack([a_src3, a_dst3], 1))
    dins = (D, 256, 256, 128)
    douts = (256, 256, 128, 128)
    modes = ("first", "fsplit", "fsplit", "esplit")

    parts = xp
    s2d = jnp.zeros((NP, 1), jnp.float32)
    for i in range(4):
        hw, sv = _tc_stage(modes[i], dins[i], douts[i], parts, s2d, Ws[i], A2s[i], NP)
        ssrc = sv[:, 0]
        sdst = sv[:, 1]
        if douts[i] == 256:
            table = hw.reshape(2 * NP, C)
            ssrc_rep = jnp.concatenate([ssrc, ssrc])
            acc, spart = _sc_layer(ssrc_rep, sdst, table, src2d, dst2d, True, NP, NCH)
        else:
            acc, spart = _sc_layer(ssrc, sdst, hw, src2d, dst2d, False, NP, NCH)
        parts = acc
        s2d = (spart[0] + spart[1]).reshape(NP, 1)

    sf_t, df_t = _tc_sfdf(parts, s2d, src_W, src_b.reshape(1, 128),
                          dst_W, dst_b.reshape(1, 128), NP)
    z = _sc_z(sf_t, df_t, src2d, dst2d, p1, EP, NCH)
    pc, msg = _tc_mlp(z, c1_W, c1_b.reshape(1, 128), p2.reshape(1, 128),
                      c2_W, c2_b.reshape(1, 2), rap, EP)
    denp, degp = _sc_den(msg.reshape(NCH, C), dst2d, NP, NCH)
    pd = _tc_den(denp, degp, NP)
    return (pc[:E], pd.reshape(NP)[:N])
